# Initial kernel scaffold; baseline (speedup 1.0000x reference)
import functools

import jax
import jax.numpy as jnp
from jax import lax
from jax.experimental import pallas as pl
from jax.experimental.pallas import tpu as pltpu
from jax.experimental.pallas import tpu_sc as plsc

NC = 2
NS = 16
LANES = 16


def _mlp1(x, W1, b1, *, grid_n=8):
    n, d_in = x.shape
    d_hid = W1.shape[1]
    blk = n // grid_n

    def body(x_ref, w_ref, b_ref, o_ref):
        h = jnp.dot(x_ref[...], w_ref[...], preferred_element_type=jnp.float32)
        h = jnp.tanh(h + b_ref[...])
        o_ref[...] = jax.nn.sigmoid(h)

    return pl.pallas_call(
        body,
        grid=(grid_n,),
        in_specs=[
            pl.BlockSpec((blk, d_in), lambda i: (i, 0)),
            pl.BlockSpec((d_in, d_hid), lambda i: (0, 0)),
            pl.BlockSpec((1, d_hid), lambda i: (0, 0)),
        ],
        out_specs=pl.BlockSpec((blk, d_hid), lambda i: (i, 0)),
        out_shape=jax.ShapeDtypeStruct((n, d_hid), jnp.float32),
    )(x, W1, b1.reshape(1, d_hid))


def _sc_aggregate(hidden, src, dst):
    n, d = hidden.shape
    e = src.shape[0]
    nw = NC * NS
    epw = e // nw
    K = 80
    n_chunks = epw // K
    rpt = n // NS
    zrows = 125
    assert e % nw == 0 and epw % K == 0 and n % NS == 0 and rpt % zrows == 0

    mesh = plsc.VectorSubcoreMesh(core_axis_name="c", subcore_axis_name="s")

    @functools.partial(
        pl.kernel,
        out_type=[
            jax.ShapeDtypeStruct((NC, n, d), jnp.float32),
            jax.ShapeDtypeStruct((NC, n, LANES), jnp.float32),
        ],
        mesh=mesh,
        scratch_types=[
            pltpu.VMEM((K,), jnp.int32),
            pltpu.VMEM((K,), jnp.int32),
            pltpu.VMEM((K, d), jnp.float32),
            pltpu.VMEM((K, LANES), jnp.float32),
            pltpu.VMEM((zrows, d), jnp.float32),
            pltpu.VMEM((rpt, LANES), jnp.float32),
            pltpu.VMEM_SHARED((n, d), jnp.float32),
            pltpu.VMEM_SHARED((n, LANES), jnp.float32),
            pltpu.SemaphoreType.DMA,
            pltpu.SemaphoreType.DMA,
        ],
    )
    def sc_kernel(hidden_hbm, src_hbm, dst_hbm, partial_hbm, deg_hbm,
                  src_v, dst_v, rows_v, ones_v, zf_v, zd_v,
                  acc_sh, deg_sh, sem0, sem1):
        c = lax.axis_index("c")
        s = lax.axis_index("s")
        wid = c * NS + s

        zero16 = jnp.zeros((LANES,), jnp.float32)
        one16 = jnp.ones((LANES,), jnp.float32)

        @pl.loop(0, zrows)
        def _(r):
            @pl.loop(0, d // LANES)
            def _(q):
                zf_v[r, pl.ds(q * LANES, LANES)] = zero16

        @pl.loop(0, rpt)
        def _(r):
            zd_v[r, :] = zero16

        @pl.loop(0, K)
        def _(r):
            ones_v[r, :] = one16

        @pl.loop(0, rpt // zrows)
        def _(k):
            pltpu.sync_copy(zf_v, acc_sh.at[pl.ds(s * rpt + k * zrows, zrows)])

        pltpu.sync_copy(zd_v, deg_sh.at[pl.ds(s * rpt, rpt)])

        plsc.subcore_barrier()

        @pl.loop(0, n_chunks)
        def _(i):
            base = wid * epw + i * K
            cp_s = pltpu.async_copy(src_hbm.at[pl.ds(base, K)], src_v, sem0)
            cp_d = pltpu.async_copy(dst_hbm.at[pl.ds(base, K)], dst_v, sem1)
            cp_s.wait()
            cp_d.wait()
            cp_g = pltpu.async_copy(hidden_hbm.at[src_v], rows_v, sem0)
            cp_deg = pltpu.async_copy(ones_v, deg_sh.at[dst_v], sem1, add=True)
            cp_g.wait()
            pltpu.sync_copy(rows_v, acc_sh.at[dst_v], add=True)
            cp_deg.wait()

        plsc.subcore_barrier()

        pltpu.sync_copy(acc_sh.at[pl.ds(s * rpt, rpt)],
                        partial_hbm.at[c, pl.ds(s * rpt, rpt)])
        pltpu.sync_copy(deg_sh.at[pl.ds(s * rpt, rpt)],
                        deg_hbm.at[c, pl.ds(s * rpt, rpt)])

    return sc_kernel(hidden, src, dst)


def _mlp2(p0, p1, d0, d1, W_out, b_out, *, grid_n=8):
    n, d_hid = p0.shape
    d_out = W_out.shape[1]
    blk = n // grid_n

    def body(p0_ref, p1_ref, d0_ref, d1_ref, w_ref, b_ref, o_ref):
        summed = p0_ref[...] + p1_ref[...]
        deg = d0_ref[...][:, 0:1] + d1_ref[...][:, 0:1]
        agg = summed / jnp.maximum(deg, 1.0)
        o_ref[...] = (
            jnp.dot(agg, w_ref[...], preferred_element_type=jnp.float32)
            + b_ref[...]
        )

    return pl.pallas_call(
        body,
        grid=(grid_n,),
        in_specs=[
            pl.BlockSpec((blk, d_hid), lambda i: (i, 0)),
            pl.BlockSpec((blk, d_hid), lambda i: (i, 0)),
            pl.BlockSpec((blk, LANES), lambda i: (i, 0)),
            pl.BlockSpec((blk, LANES), lambda i: (i, 0)),
            pl.BlockSpec((d_hid, d_out), lambda i: (0, 0)),
            pl.BlockSpec((1, d_out), lambda i: (0, 0)),
        ],
        out_specs=pl.BlockSpec((blk, d_out), lambda i: (i, 0)),
        out_shape=jax.ShapeDtypeStruct((n, d_out), jnp.float32),
    )(p0, p1, d0, d1, W_out, b_out.reshape(1, d_out))


def kernel(node_features, edge_index, W1, b1, W_out, b_out):
    src = edge_index[0].astype(jnp.int32)
    dst = edge_index[1].astype(jnp.int32)
    hidden = _mlp1(node_features, W1, b1)
    partial, degp = _sc_aggregate(hidden, src, dst)
    return _mlp2(partial[0], partial[1], degp[0], degp[1], W_out, b_out)


# speedup vs baseline: 4.7745x; 4.7745x over previous
import functools

import jax
import jax.numpy as jnp
from jax import lax
from jax.experimental import pallas as pl
from jax.experimental.pallas import tpu as pltpu
from jax.experimental.pallas import tpu_sc as plsc

NC = 2
NS = 16
LANES = 16


def _mlp1(x, W1, b1, *, grid_n=10):
    n, d_in = x.shape
    d_hid = W1.shape[1]
    dh = d_hid // 2
    blk = n // grid_n

    def body(x_ref, w_ref, b_ref, h0_ref, h1_ref):
        h = jnp.dot(x_ref[...], w_ref[...], preferred_element_type=jnp.float32)
        h = jax.nn.sigmoid(jnp.tanh(h + b_ref[...]))
        h0_ref[...] = h[:, :dh]
        h1_ref[...] = h[:, dh:]

    return pl.pallas_call(
        body,
        grid=(grid_n,),
        in_specs=[
            pl.BlockSpec((blk, d_in), lambda i: (i, 0)),
            pl.BlockSpec((d_in, d_hid), lambda i: (0, 0)),
            pl.BlockSpec((1, d_hid), lambda i: (0, 0)),
        ],
        out_specs=[
            pl.BlockSpec((blk, dh), lambda i: (i, 0)),
            pl.BlockSpec((blk, dh), lambda i: (i, 0)),
        ],
        out_shape=[
            jax.ShapeDtypeStruct((n, dh), jnp.float32),
            jax.ShapeDtypeStruct((n, dh), jnp.float32),
        ],
    )(x, W1, b1.reshape(1, d_hid))


def _sc_aggregate(h0, h1, src, dst):
    n, dh = h0.shape
    e = src.shape[0]
    ept = e // NS
    K = 80
    n_chunks = ept // K
    half = n_chunks // 2
    zrows = 128
    rpt = -(-n // (NS * zrows)) * zrows
    n_pad = NS * rpt
    assert e % NS == 0 and ept % K == 0 and n_chunks % 2 == 0

    mesh = plsc.VectorSubcoreMesh(core_axis_name="c", subcore_axis_name="s")

    @functools.partial(
        pl.kernel,
        out_type=[
            jax.ShapeDtypeStruct((NC, n_pad, dh), jnp.float32),
            jax.ShapeDtypeStruct((NC, n_pad, LANES), jnp.float32),
        ],
        mesh=mesh,
        compiler_params=pltpu.CompilerParams(use_tc_tiling_on_sc=False),
        scratch_types=[
            pltpu.VMEM((K,), jnp.int32),
            pltpu.VMEM((K,), jnp.int32),
            pltpu.VMEM((K, dh), jnp.float32),
            pltpu.VMEM((K, LANES), jnp.float32),
            pltpu.VMEM((zrows, dh), jnp.float32),
            pltpu.VMEM((rpt, LANES), jnp.float32),
            pltpu.VMEM_SHARED((n_pad, dh), jnp.float32),
            pltpu.VMEM_SHARED((n_pad, LANES), jnp.float32),
            pltpu.SemaphoreType.DMA,
            pltpu.SemaphoreType.DMA,
        ],
    )
    def sc_kernel(h0_hbm, h1_hbm, src_hbm, dst_hbm, partial_hbm, deg_hbm,
                  src_v, dst_v, rows_v, ones_v, zf_v, zd_v,
                  acc_sh, deg_sh, sem0, sem1):
        c = lax.axis_index("c")
        s = lax.axis_index("s")

        zero16 = jnp.zeros((LANES,), jnp.float32)
        one16 = jnp.ones((LANES,), jnp.float32)

        @pl.loop(0, zrows)
        def _(r):
            @pl.loop(0, dh // LANES)
            def _(q):
                zf_v[r, pl.ds(q * LANES, LANES)] = zero16

        @pl.loop(0, rpt)
        def _(r):
            zd_v[r, :] = zero16

        @pl.loop(0, K)
        def _(r):
            ones_v[r, :] = one16

        @pl.loop(0, rpt // zrows)
        def _(k):
            pltpu.sync_copy(zf_v, acc_sh.at[pl.ds(s * rpt + k * zrows, zrows)])

        pltpu.sync_copy(zd_v, deg_sh.at[pl.ds(s * rpt, rpt)])

        plsc.subcore_barrier()

        @pl.loop(0, n_chunks)
        def _(i):
            base = s * ept + i * K
            cp_s = pltpu.async_copy(src_hbm.at[pl.ds(base, K)], src_v, sem0)
            cp_d = pltpu.async_copy(dst_hbm.at[pl.ds(base, K)], dst_v, sem1)
            cp_s.wait()
            cp_d.wait()

            @pl.when(c == 0)
            def _():
                pltpu.sync_copy(h0_hbm.at[src_v], rows_v)

            @pl.when(c == 1)
            def _():
                pltpu.sync_copy(h1_hbm.at[src_v], rows_v)

            @pl.when(jnp.where(c == 0, i < half, i >= half))
            def _():
                pltpu.sync_copy(ones_v, deg_sh.at[dst_v], add=True)

            pltpu.sync_copy(rows_v, acc_sh.at[dst_v], add=True)

        plsc.subcore_barrier()

        pltpu.sync_copy(acc_sh.at[pl.ds(s * rpt, rpt)],
                        partial_hbm.at[c, pl.ds(s * rpt, rpt)])
        pltpu.sync_copy(deg_sh.at[pl.ds(s * rpt, rpt)],
                        deg_hbm.at[c, pl.ds(s * rpt, rpt)])

    return sc_kernel(h0, h1, src, dst)


def _mlp2(partial, degp, W_out, b_out, n, *, grid_n=10):
    dh = partial.shape[2]
    d_hid = NC * dh
    d_out = W_out.shape[1]
    blk = n // grid_n

    def body(p_ref, d_ref, w_ref, b_ref, o_ref):
        summed = jnp.concatenate([p_ref[0], p_ref[1]], axis=1)
        deg = d_ref[0][:, 0:1] + d_ref[1][:, 0:1]
        agg = summed / jnp.maximum(deg, 1.0)
        o_ref[...] = (
            jnp.dot(agg, w_ref[...], preferred_element_type=jnp.float32)
            + b_ref[...]
        )

    return pl.pallas_call(
        body,
        grid=(grid_n,),
        in_specs=[
            pl.BlockSpec((NC, blk, dh), lambda i: (0, i, 0)),
            pl.BlockSpec((NC, blk, LANES), lambda i: (0, i, 0)),
            pl.BlockSpec((d_hid, d_out), lambda i: (0, 0)),
            pl.BlockSpec((1, d_out), lambda i: (0, 0)),
        ],
        out_specs=pl.BlockSpec((blk, d_out), lambda i: (i, 0)),
        out_shape=jax.ShapeDtypeStruct((n, d_out), jnp.float32),
    )(partial, degp, W_out, b_out.reshape(1, d_out))


def kernel(node_features, edge_index, W1, b1, W_out, b_out):
    src = edge_index[0].astype(jnp.int32)
    dst = edge_index[1].astype(jnp.int32)
    h0, h1 = _mlp1(node_features, W1, b1)
    partial, degp = _sc_aggregate(h0, h1, src, dst)
    return _mlp2(partial, degp, W_out, b_out, node_features.shape[0])


# speedup vs baseline: 9.9394x; 2.0818x over previous
import functools

import jax
import jax.numpy as jnp
from jax import lax
from jax.experimental import pallas as pl
from jax.experimental.pallas import tpu as pltpu
from jax.experimental.pallas import tpu_sc as plsc

NC = 2
NS = 16
LANES = 16


def _mlp1(x, W1, b1, *, grid_n=10):
    n, d_in = x.shape
    d_hid = W1.shape[1]
    dh = d_hid // 2
    blk = n // grid_n

    def body(x_ref, w_ref, b_ref, h0_ref, h1_ref):
        h = jnp.dot(x_ref[...], w_ref[...], preferred_element_type=jnp.float32)
        h = jax.nn.sigmoid(jnp.tanh(h + b_ref[...]))
        h0_ref[...] = h[:, :dh]
        h1_ref[...] = h[:, dh:]

    return pl.pallas_call(
        body,
        grid=(grid_n,),
        in_specs=[
            pl.BlockSpec((blk, d_in), lambda i: (i, 0)),
            pl.BlockSpec((d_in, d_hid), lambda i: (0, 0)),
            pl.BlockSpec((1, d_hid), lambda i: (0, 0)),
        ],
        out_specs=[
            pl.BlockSpec((blk, dh), lambda i: (i, 0)),
            pl.BlockSpec((blk, dh), lambda i: (i, 0)),
        ],
        out_shape=[
            jax.ShapeDtypeStruct((n, dh), jnp.float32),
            jax.ShapeDtypeStruct((n, dh), jnp.float32),
        ],
    )(x, W1, b1.reshape(1, d_hid))


K = 80
G = 5


def _sc_aggregate(h0, h1, edges3):
    n, dh = h0.shape
    total_chunks = edges3.shape[0]
    e = total_chunks * K
    ept = e // NS
    n_chunks = ept // K
    half = n_chunks // 2
    zrows = 128
    rpt = -(-n // (NS * zrows)) * zrows
    n_pad = NS * rpt
    ngroups = n_chunks // G
    assert e % NS == 0 and ept % K == 0 and n_chunks % (2 * G) == 0

    mesh = plsc.VectorSubcoreMesh(core_axis_name="c", subcore_axis_name="s")

    @functools.partial(
        pl.kernel,
        out_type=[
            jax.ShapeDtypeStruct((NC, n_pad, dh), jnp.float32),
            jax.ShapeDtypeStruct((NC, n_pad, LANES), jnp.float32),
        ],
        mesh=mesh,
        compiler_params=pltpu.CompilerParams(use_tc_tiling_on_sc=False),
        scratch_types=(
            [pltpu.VMEM((2, K), jnp.int32) for _ in range(G)]
            + [pltpu.VMEM((K, dh), jnp.float32) for _ in range(G)]
            + [
                pltpu.VMEM((K, LANES), jnp.float32),
                pltpu.VMEM((zrows, dh), jnp.float32),
                pltpu.VMEM((rpt, LANES), jnp.float32),
                pltpu.VMEM_SHARED((n_pad, dh), jnp.float32),
                pltpu.VMEM_SHARED((n_pad, LANES), jnp.float32),
            ]
            + [pltpu.SemaphoreType.DMA for _ in range(4 * G)]
        ),
    )
    def sc_kernel(h0_hbm, h1_hbm, edges_hbm, partial_hbm, deg_hbm, *scr):
        idx_v = scr[:G]
        rows_v = scr[G:2 * G]
        ones_v, zf_v, zd_v, acc_sh, deg_sh = scr[2 * G:2 * G + 5]
        sems = scr[2 * G + 5:]
        isem, gsem, ssem, dsem = (sems[i * G:(i + 1) * G] for i in range(4))

        c = lax.axis_index("c")
        s = lax.axis_index("s")

        zero16 = jnp.zeros((LANES,), jnp.float32)
        one16 = jnp.ones((LANES,), jnp.float32)

        @pl.loop(0, zrows)
        def _(r):
            @pl.loop(0, dh // LANES)
            def _(q):
                zf_v[r, pl.ds(q * LANES, LANES)] = zero16

        @pl.loop(0, rpt)
        def _(r):
            zd_v[r, :] = zero16

        @pl.loop(0, K)
        def _(r):
            ones_v[r, :] = one16

        @pl.loop(0, rpt // zrows)
        def _(k):
            pltpu.sync_copy(zf_v, acc_sh.at[pl.ds(s * rpt + k * zrows, zrows)])

        pltpu.sync_copy(zd_v, deg_sh.at[pl.ds(s * rpt, rpt)])

        plsc.subcore_barrier()

        def chunk_id(g, b):
            return s * n_chunks + g * G + b

        for b in range(G):
            pltpu.async_copy(edges_hbm.at[chunk_id(0, b)], idx_v[b], isem[b])

        @pl.loop(0, ngroups)
        def _(g):
            for b in range(G):
                pltpu.make_async_copy(
                    edges_hbm.at[chunk_id(g, b)], idx_v[b], isem[b]).wait()

                @pl.when(c == 0)
                def _(b=b):
                    pltpu.async_copy(
                        h0_hbm.at[idx_v[b].at[0]], rows_v[b], gsem[b])

                @pl.when(c == 1)
                def _(b=b):
                    pltpu.async_copy(
                        h1_hbm.at[idx_v[b].at[0]], rows_v[b], gsem[b])

            scatters = []
            for b in range(G):
                pltpu.make_async_copy(
                    h0_hbm.at[idx_v[b].at[0]], rows_v[b], gsem[b]).wait()
                scatters.append(pltpu.async_copy(
                    rows_v[b], acc_sh.at[idx_v[b].at[1]], ssem[b], add=True))
                i = g * G + b
                @pl.when(jnp.where(c == 0, i < half, i >= half))
                def _(b=b, i=i):
                    pltpu.async_copy(
                        ones_v, deg_sh.at[idx_v[b].at[1]], dsem[b], add=True)

            for b in range(G):
                scatters[b].wait()
                i = g * G + b

                @pl.when(jnp.where(c == 0, i < half, i >= half))
                def _(b=b, i=i):
                    pltpu.make_async_copy(
                        ones_v, deg_sh.at[idx_v[b].at[1]], dsem[b]).wait()

                @pl.when(g + 1 < ngroups)
                def _(b=b):
                    pltpu.async_copy(
                        edges_hbm.at[chunk_id(g + 1, b)], idx_v[b], isem[b])

        plsc.subcore_barrier()

        pltpu.sync_copy(acc_sh.at[pl.ds(s * rpt, rpt)],
                        partial_hbm.at[c, pl.ds(s * rpt, rpt)])
        pltpu.sync_copy(deg_sh.at[pl.ds(s * rpt, rpt)],
                        deg_hbm.at[c, pl.ds(s * rpt, rpt)])

    return sc_kernel(h0, h1, edges3)


def _mlp2(partial, degp, W_out, b_out, n, *, grid_n=10):
    dh = partial.shape[2]
    d_hid = NC * dh
    d_out = W_out.shape[1]
    blk = n // grid_n

    def body(p_ref, d_ref, w_ref, b_ref, o_ref):
        summed = jnp.concatenate([p_ref[0], p_ref[1]], axis=1)
        deg = d_ref[0][:, 0:1] + d_ref[1][:, 0:1]
        agg = summed / jnp.maximum(deg, 1.0)
        o_ref[...] = (
            jnp.dot(agg, w_ref[...], preferred_element_type=jnp.float32)
            + b_ref[...]
        )

    return pl.pallas_call(
        body,
        grid=(grid_n,),
        in_specs=[
            pl.BlockSpec((NC, blk, dh), lambda i: (0, i, 0)),
            pl.BlockSpec((NC, blk, LANES), lambda i: (0, i, 0)),
            pl.BlockSpec((d_hid, d_out), lambda i: (0, 0)),
            pl.BlockSpec((1, d_out), lambda i: (0, 0)),
        ],
        out_specs=pl.BlockSpec((blk, d_out), lambda i: (i, 0)),
        out_shape=jax.ShapeDtypeStruct((n, d_out), jnp.float32),
    )(partial, degp, W_out, b_out.reshape(1, d_out))


def kernel(node_features, edge_index, W1, b1, W_out, b_out):
    e = edge_index.shape[1]
    edges3 = jnp.transpose(
        edge_index.astype(jnp.int32).reshape(2, e // K, K), (1, 0, 2))
    h0, h1 = _mlp1(node_features, W1, b1)
    partial, degp = _sc_aggregate(h0, h1, edges3)
    return _mlp2(partial, degp, W_out, b_out, node_features.shape[0])


# speedup vs baseline: 11.8607x; 1.1933x over previous
import functools

import jax
import jax.numpy as jnp
from jax import lax
from jax.experimental import pallas as pl
from jax.experimental.pallas import tpu as pltpu
from jax.experimental.pallas import tpu_sc as plsc

NC = 2
NS = 16
LANES = 16


def _mlp1(x, W1, b1, *, grid_n=5):
    n, d_in = x.shape
    d_hid = W1.shape[1]
    blk = n // grid_n

    def body(x_ref, w_ref, b_ref, o_ref):
        h = jnp.dot(x_ref[...], w_ref[...], preferred_element_type=jnp.float32)
        o_ref[...] = jax.nn.sigmoid(jnp.tanh(h + b_ref[...]))

    return pl.pallas_call(
        body,
        grid=(grid_n,),
        in_specs=[
            pl.BlockSpec((blk, d_in), lambda i: (i, 0)),
            pl.BlockSpec((d_in, d_hid), lambda i: (0, 0)),
            pl.BlockSpec((1, d_hid), lambda i: (0, 0)),
        ],
        out_specs=pl.BlockSpec((blk, d_hid), lambda i: (i, 0)),
        out_shape=jax.ShapeDtypeStruct((n, d_hid), jnp.float32),
    )(x, W1, b1.reshape(1, d_hid))


K = 80
G = 5


def _sc_aggregate(hidden2, idx3):
    n2, dh = hidden2.shape
    n = n2 // NC
    e = idx3.shape[1]
    ept = e // NS
    n_chunks = ept // K
    half = n_chunks // 2
    zrows = 128
    rpt = -(-n // (NS * zrows)) * zrows
    n_pad = NS * rpt
    ngroups = n_chunks // G
    assert e % NS == 0 and ept % K == 0 and n_chunks % G == 0

    mesh = plsc.VectorSubcoreMesh(core_axis_name="c", subcore_axis_name="s")

    @functools.partial(
        pl.kernel,
        out_type=[
            jax.ShapeDtypeStruct((n_pad, NC * dh), jnp.float32),
            jax.ShapeDtypeStruct((NC, n_pad, LANES), jnp.float32),
        ],
        mesh=mesh,
        compiler_params=pltpu.CompilerParams(use_tc_tiling_on_sc=False),
        scratch_types=(
            [pltpu.VMEM((K,), jnp.int32) for _ in range(2 * G)]
            + [pltpu.VMEM((K, dh), jnp.float32) for _ in range(G)]
            + [
                pltpu.VMEM((K, LANES), jnp.float32),
                pltpu.VMEM((zrows, dh), jnp.float32),
                pltpu.VMEM((rpt, LANES), jnp.float32),
                pltpu.VMEM_SHARED((n_pad, dh), jnp.float32),
                pltpu.VMEM_SHARED((n_pad, LANES), jnp.float32),
            ]
            + [pltpu.SemaphoreType.DMA for _ in range(4 * G)]
        ),
    )
    def sc_kernel(h_hbm, idx_hbm, partial_hbm, deg_hbm, *scr):
        src_v = scr[:G]
        dst_v = scr[G:2 * G]
        rows_v = scr[2 * G:3 * G]
        ones_v, zf_v, zd_v, acc_sh, deg_sh = scr[3 * G:3 * G + 5]
        sems = scr[3 * G + 5:]
        isem, gsem, ssem, dsem = (sems[i * G:(i + 1) * G] for i in range(4))

        c = lax.axis_index("c")
        s = lax.axis_index("s")

        zero16 = jnp.zeros((LANES,), jnp.float32)
        one16 = jnp.ones((LANES,), jnp.float32)

        @pl.loop(0, zrows)
        def _(r):
            @pl.loop(0, dh // LANES)
            def _(q):
                zf_v[r, pl.ds(q * LANES, LANES)] = zero16

        @pl.loop(0, rpt)
        def _(r):
            zd_v[r, :] = zero16

        @pl.loop(0, K)
        def _(r):
            ones_v[r, :] = one16

        @pl.loop(0, rpt // zrows)
        def _(k):
            pltpu.sync_copy(zf_v, acc_sh.at[pl.ds(s * rpt + k * zrows, zrows)])

        pltpu.sync_copy(zd_v, deg_sh.at[pl.ds(s * rpt, rpt)])

        plsc.subcore_barrier()

        def cbase(g, b):
            return s * ept + (g * G + b) * K

        for b in range(G):
            pltpu.async_copy(idx_hbm.at[c, pl.ds(cbase(0, b), K)], src_v[b], isem[b])
            pltpu.async_copy(idx_hbm.at[2, pl.ds(cbase(0, b), K)], dst_v[b], isem[b])

        @pl.loop(0, ngroups)
        def _(g):
            for b in range(G):
                pltpu.make_async_copy(
                    idx_hbm.at[c, pl.ds(cbase(g, b), K)], src_v[b], isem[b]).wait()
                pltpu.make_async_copy(
                    idx_hbm.at[2, pl.ds(cbase(g, b), K)], dst_v[b], isem[b]).wait()
                pltpu.async_copy(h_hbm.at[src_v[b]], rows_v[b], gsem[b])

            scatters = []
            for b in range(G):
                pltpu.make_async_copy(
                    h_hbm.at[src_v[b]], rows_v[b], gsem[b]).wait()
                scatters.append(pltpu.async_copy(
                    rows_v[b], acc_sh.at[dst_v[b]], ssem[b], add=True))
                i = g * G + b
                @pl.when(jnp.where(c == 0, i < half, i >= half))
                def _(b=b, i=i):
                    pltpu.async_copy(
                        ones_v, deg_sh.at[dst_v[b]], dsem[b], add=True)

            for b in range(G):
                scatters[b].wait()
                i = g * G + b

                @pl.when(jnp.where(c == 0, i < half, i >= half))
                def _(b=b, i=i):
                    pltpu.make_async_copy(
                        ones_v, deg_sh.at[dst_v[b]], dsem[b]).wait()

                @pl.when(g + 1 < ngroups)
                def _(b=b):
                    pltpu.async_copy(
                        idx_hbm.at[c, pl.ds(cbase(g + 1, b), K)], src_v[b], isem[b])
                    pltpu.async_copy(
                        idx_hbm.at[2, pl.ds(cbase(g + 1, b), K)], dst_v[b], isem[b])

        plsc.subcore_barrier()

        pltpu.sync_copy(acc_sh.at[pl.ds(s * rpt, rpt)],
                        partial_hbm.at[pl.ds(s * rpt, rpt), pl.ds(c * dh, dh)])
        pltpu.sync_copy(deg_sh.at[pl.ds(s * rpt, rpt)],
                        deg_hbm.at[c, pl.ds(s * rpt, rpt)])

    return sc_kernel(hidden2, idx3)


def _mlp2(partial, degp, W_out, b_out, n, *, grid_n=5):
    d_hid = partial.shape[1]
    d_out = W_out.shape[1]
    blk = n // grid_n

    def body(p_ref, d_ref, w_ref, b_ref, o_ref):
        deg = d_ref[0][:, 0:1] + d_ref[1][:, 0:1]
        agg = p_ref[...] / jnp.maximum(deg, 1.0)
        o_ref[...] = (
            jnp.dot(agg, w_ref[...], preferred_element_type=jnp.float32)
            + b_ref[...]
        )

    return pl.pallas_call(
        body,
        grid=(grid_n,),
        in_specs=[
            pl.BlockSpec((blk, d_hid), lambda i: (i, 0)),
            pl.BlockSpec((NC, blk, LANES), lambda i: (0, i, 0)),
            pl.BlockSpec((d_hid, d_out), lambda i: (0, 0)),
            pl.BlockSpec((1, d_out), lambda i: (0, 0)),
        ],
        out_specs=pl.BlockSpec((blk, d_out), lambda i: (i, 0)),
        out_shape=jax.ShapeDtypeStruct((n, d_out), jnp.float32),
    )(partial, degp, W_out, b_out.reshape(1, d_out))


def kernel(node_features, edge_index, W1, b1, W_out, b_out):
    n, d_hid = node_features.shape[0], W1.shape[1]
    ei = edge_index.astype(jnp.int32)
    idx3 = jnp.stack([2 * ei[0], 2 * ei[0] + 1, ei[1]])
    hidden = _mlp1(node_features, W1, b1)
    hidden2 = hidden.reshape(NC * n, d_hid // NC)
    partial, degp = _sc_aggregate(hidden2, idx3)
    return _mlp2(partial, degp, W_out, b_out, n)


# speedup vs baseline: 13.8671x; 1.1692x over previous
import functools

import jax
import jax.numpy as jnp
from jax import lax
from jax.experimental import pallas as pl
from jax.experimental.pallas import tpu as pltpu
from jax.experimental.pallas import tpu_sc as plsc

NC = 2
NS = 16
LANES = 16


def _mlp1(x, W1, b1, *, grid_n=5):
    n, d_in = x.shape
    d_hid = W1.shape[1]
    blk = n // grid_n

    def body(x_ref, w_ref, b_ref, o_ref):
        h = jnp.dot(x_ref[...], w_ref[...], preferred_element_type=jnp.float32)
        o_ref[...] = jax.nn.sigmoid(jnp.tanh(h + b_ref[...]))

    return pl.pallas_call(
        body,
        grid=(grid_n,),
        in_specs=[
            pl.BlockSpec((blk, d_in), lambda i: (i, 0)),
            pl.BlockSpec((d_in, d_hid), lambda i: (0, 0)),
            pl.BlockSpec((1, d_hid), lambda i: (0, 0)),
        ],
        out_specs=pl.BlockSpec((blk, d_hid), lambda i: (i, 0)),
        out_shape=jax.ShapeDtypeStruct((n, d_hid), jnp.float32),
    )(x, W1, b1.reshape(1, d_hid))


K = 80
G = 5


def _sc_aggregate(hidden2, idx3):
    n2, dh = hidden2.shape
    n = n2 // NC
    e = idx3.shape[1]
    ept = e // NS
    n_chunks = ept // K
    half = n_chunks // 2
    zrows = 128
    rpt = -(-n // (NS * zrows)) * zrows
    n_pad = NS * rpt
    ngroups = n_chunks // G
    assert e % NS == 0 and ept % K == 0 and n_chunks % G == 0 and ngroups % 2 == 0

    mesh = plsc.VectorSubcoreMesh(core_axis_name="c", subcore_axis_name="s")

    @functools.partial(
        pl.kernel,
        out_type=[
            jax.ShapeDtypeStruct((n_pad, NC * dh), jnp.float32),
            jax.ShapeDtypeStruct((NC, n_pad, LANES), jnp.float32),
        ],
        mesh=mesh,
        compiler_params=pltpu.CompilerParams(use_tc_tiling_on_sc=False),
        scratch_types=(
            [pltpu.VMEM((K,), jnp.int32) for _ in range(4 * G)]
            + [pltpu.VMEM((K, dh), jnp.float32) for _ in range(G)]
            + [
                pltpu.VMEM((K, LANES), jnp.float32),
                pltpu.VMEM((zrows, dh), jnp.float32),
                pltpu.VMEM((rpt, LANES), jnp.float32),
                pltpu.VMEM_SHARED((n_pad, dh), jnp.float32),
                pltpu.VMEM_SHARED((n_pad, LANES), jnp.float32),
            ]
            + [pltpu.SemaphoreType.DMA for _ in range(5 * G)]
        ),
    )
    def sc_kernel(h_hbm, idx_hbm, partial_hbm, deg_hbm, *scr):
        srcA = scr[:G]
        dstA = scr[G:2 * G]
        srcB = scr[2 * G:3 * G]
        dstB = scr[3 * G:4 * G]
        rows_v = scr[4 * G:5 * G]
        ones_v, zf_v, zd_v, acc_sh, deg_sh = scr[5 * G:5 * G + 5]
        sems = scr[5 * G + 5:]
        isemA, isemB, gsem, ssem, dsem = (
            sems[i * G:(i + 1) * G] for i in range(5))

        c = lax.axis_index("c")
        s = lax.axis_index("s")

        zero16 = jnp.zeros((LANES,), jnp.float32)
        one16 = jnp.ones((LANES,), jnp.float32)

        @pl.loop(0, zrows)
        def _(r):
            @pl.loop(0, dh // LANES)
            def _(q):
                zf_v[r, pl.ds(q * LANES, LANES)] = zero16

        @pl.loop(0, rpt)
        def _(r):
            zd_v[r, :] = zero16

        @pl.loop(0, K)
        def _(r):
            ones_v[r, :] = one16

        @pl.loop(0, rpt // zrows)
        def _(k):
            pltpu.sync_copy(zf_v, acc_sh.at[pl.ds(s * rpt + k * zrows, zrows)])

        pltpu.sync_copy(zd_v, deg_sh.at[pl.ds(s * rpt, rpt)])

        plsc.subcore_barrier()

        def cbase(g, b):
            return s * ept + (g * G + b) * K

        def issue_idx(g, b, sv, dv, sem):
            pltpu.async_copy(idx_hbm.at[c, pl.ds(cbase(g, b), K)], sv[b], sem[b])
            pltpu.async_copy(idx_hbm.at[2, pl.ds(cbase(g, b), K)], dv[b], sem[b])

        def wait_idx(g, b, sv, dv, sem):
            pltpu.make_async_copy(
                idx_hbm.at[c, pl.ds(cbase(g, b), K)], sv[b], sem[b]).wait()
            pltpu.make_async_copy(
                idx_hbm.at[2, pl.ds(cbase(g, b), K)], dv[b], sem[b]).wait()

        for b in range(G):
            issue_idx(0, b, srcA, dstA, isemA)
            issue_idx(1, b, srcB, dstB, isemB)
        for b in range(G):
            wait_idx(0, b, srcA, dstA, isemA)
            pltpu.async_copy(h_hbm.at[srcA[b]], rows_v[b], gsem[b])

        def subgroup(g, cur, nxt):
            (csrc, cdst, cisem), (nsrc, ndst, nisem) = cur, nxt
            scatters = []
            for b in range(G):
                pltpu.make_async_copy(
                    h_hbm.at[csrc[b]], rows_v[b], gsem[b]).wait()
                scatters.append(pltpu.async_copy(
                    rows_v[b], acc_sh.at[cdst[b]], ssem[b], add=True))
                i = g * G + b
                @pl.when(jnp.where(c == 0, i < half, i >= half))
                def _(b=b, i=i):
                    pltpu.async_copy(
                        ones_v, deg_sh.at[cdst[b]], dsem[b], add=True)

            for b in range(G):
                scatters[b].wait()
                i = g * G + b

                @pl.when(jnp.where(c == 0, i < half, i >= half))
                def _(b=b, i=i):
                    pltpu.make_async_copy(
                        ones_v, deg_sh.at[cdst[b]], dsem[b]).wait()

                @pl.when(g + 1 < ngroups)
                def _(b=b):
                    wait_idx(g + 1, b, nsrc, ndst, nisem)
                    pltpu.async_copy(h_hbm.at[nsrc[b]], rows_v[b], gsem[b])

                @pl.when(g + 2 < ngroups)
                def _(b=b):
                    issue_idx(g + 2, b, csrc, cdst, cisem)

        ringA = (srcA, dstA, isemA)
        ringB = (srcB, dstB, isemB)

        @pl.loop(0, ngroups // 2)
        def _(t):
            subgroup(2 * t, ringA, ringB)
            subgroup(2 * t + 1, ringB, ringA)

        plsc.subcore_barrier()

        pltpu.sync_copy(acc_sh.at[pl.ds(s * rpt, rpt)],
                        partial_hbm.at[pl.ds(s * rpt, rpt), pl.ds(c * dh, dh)])
        pltpu.sync_copy(deg_sh.at[pl.ds(s * rpt, rpt)],
                        deg_hbm.at[c, pl.ds(s * rpt, rpt)])

    return sc_kernel(hidden2, idx3)


def _mlp2(partial, degp, W_out, b_out, n, *, grid_n=5):
    d_hid = partial.shape[1]
    d_out = W_out.shape[1]
    blk = n // grid_n

    def body(p_ref, d_ref, w_ref, b_ref, o_ref):
        deg = d_ref[0][:, 0:1] + d_ref[1][:, 0:1]
        agg = p_ref[...] / jnp.maximum(deg, 1.0)
        o_ref[...] = (
            jnp.dot(agg, w_ref[...], preferred_element_type=jnp.float32)
            + b_ref[...]
        )

    return pl.pallas_call(
        body,
        grid=(grid_n,),
        in_specs=[
            pl.BlockSpec((blk, d_hid), lambda i: (i, 0)),
            pl.BlockSpec((NC, blk, LANES), lambda i: (0, i, 0)),
            pl.BlockSpec((d_hid, d_out), lambda i: (0, 0)),
            pl.BlockSpec((1, d_out), lambda i: (0, 0)),
        ],
        out_specs=pl.BlockSpec((blk, d_out), lambda i: (i, 0)),
        out_shape=jax.ShapeDtypeStruct((n, d_out), jnp.float32),
    )(partial, degp, W_out, b_out.reshape(1, d_out))


def kernel(node_features, edge_index, W1, b1, W_out, b_out):
    n, d_hid = node_features.shape[0], W1.shape[1]
    ei = edge_index.astype(jnp.int32)
    idx3 = jnp.stack([2 * ei[0], 2 * ei[0] + 1, ei[1]])
    hidden = _mlp1(node_features, W1, b1)
    hidden2 = hidden.reshape(NC * n, d_hid // NC)
    partial, degp = _sc_aggregate(hidden2, idx3)
    return _mlp2(partial, degp, W_out, b_out, n)


# speedup vs baseline: 14.1823x; 1.0227x over previous
import functools

import jax
import jax.numpy as jnp
from jax import lax
from jax.experimental import pallas as pl
from jax.experimental.pallas import tpu as pltpu
from jax.experimental.pallas import tpu_sc as plsc

NC = 2
NS = 16
LANES = 16


def _mlp1(x, W1, b1, *, grid_n=2):
    n, d_in = x.shape
    d_hid = W1.shape[1]
    blk = n // grid_n

    def body(x_ref, w_ref, b_ref, o_ref):
        h = jnp.dot(x_ref[...], w_ref[...], preferred_element_type=jnp.float32)
        o_ref[...] = jax.nn.sigmoid(jnp.tanh(h + b_ref[...]))

    return pl.pallas_call(
        body,
        grid=(grid_n,),
        in_specs=[
            pl.BlockSpec((blk, d_in), lambda i: (i, 0)),
            pl.BlockSpec((d_in, d_hid), lambda i: (0, 0)),
            pl.BlockSpec((1, d_hid), lambda i: (0, 0)),
        ],
        out_specs=pl.BlockSpec((blk, d_hid), lambda i: (i, 0)),
        out_shape=jax.ShapeDtypeStruct((n, d_hid), jnp.float32),
    )(x, W1, b1.reshape(1, d_hid))


K = 80
G = 5


def _sc_aggregate(hidden2, idx3):
    n2, dh = hidden2.shape
    n = n2 // NC
    e = idx3.shape[1]
    ept = e // NS
    n_chunks = ept // K
    half = n_chunks // 2
    zrows = 128
    rpt = -(-n // (NS * zrows)) * zrows
    n_pad = NS * rpt
    ngroups = n_chunks // G
    assert e % NS == 0 and ept % K == 0 and n_chunks % G == 0 and ngroups % 2 == 0

    mesh = plsc.VectorSubcoreMesh(core_axis_name="c", subcore_axis_name="s")

    @functools.partial(
        pl.kernel,
        out_type=[
            jax.ShapeDtypeStruct((n_pad, NC * dh), jnp.float32),
            jax.ShapeDtypeStruct((NC, n_pad, LANES), jnp.float32),
        ],
        mesh=mesh,
        compiler_params=pltpu.CompilerParams(use_tc_tiling_on_sc=False),
        scratch_types=(
            [pltpu.VMEM((K,), jnp.int32) for _ in range(4 * G)]
            + [pltpu.VMEM((K, dh), jnp.float32) for _ in range(G)]
            + [
                pltpu.VMEM((K, LANES), jnp.float32),
                pltpu.VMEM((zrows, dh), jnp.float32),
                pltpu.VMEM((rpt, LANES), jnp.float32),
                pltpu.VMEM_SHARED((n_pad, dh), jnp.float32),
                pltpu.VMEM_SHARED((n_pad, LANES), jnp.float32),
            ]
            + [pltpu.SemaphoreType.DMA for _ in range(5 * G)]
        ),
    )
    def sc_kernel(h_hbm, idx_hbm, partial_hbm, deg_hbm, *scr):
        srcA = scr[:G]
        dstA = scr[G:2 * G]
        srcB = scr[2 * G:3 * G]
        dstB = scr[3 * G:4 * G]
        rows_v = scr[4 * G:5 * G]
        ones_v, zf_v, zd_v, acc_sh, deg_sh = scr[5 * G:5 * G + 5]
        sems = scr[5 * G + 5:]
        isemA, isemB, gsem, ssem, dsem = (
            sems[i * G:(i + 1) * G] for i in range(5))

        c = lax.axis_index("c")
        s = lax.axis_index("s")

        zero16 = jnp.zeros((LANES,), jnp.float32)
        one16 = jnp.ones((LANES,), jnp.float32)

        @pl.loop(0, zrows)
        def _(r):
            @pl.loop(0, dh // LANES)
            def _(q):
                zf_v[r, pl.ds(q * LANES, LANES)] = zero16

        @pl.loop(0, rpt)
        def _(r):
            zd_v[r, :] = zero16

        @pl.loop(0, K)
        def _(r):
            ones_v[r, :] = one16

        @pl.loop(0, rpt // zrows)
        def _(k):
            pltpu.sync_copy(zf_v, acc_sh.at[pl.ds(s * rpt + k * zrows, zrows)])

        pltpu.sync_copy(zd_v, deg_sh.at[pl.ds(s * rpt, rpt)])

        plsc.subcore_barrier()

        def cbase(g, b):
            return s * ept + (g * G + b) * K

        def issue_idx(g, b, sv, dv, sem):
            pltpu.async_copy(idx_hbm.at[c, pl.ds(cbase(g, b), K)], sv[b], sem[b])
            pltpu.async_copy(idx_hbm.at[2, pl.ds(cbase(g, b), K)], dv[b], sem[b])

        def wait_idx(g, b, sv, dv, sem):
            pltpu.make_async_copy(
                idx_hbm.at[c, pl.ds(cbase(g, b), K)], sv[b], sem[b]).wait()
            pltpu.make_async_copy(
                idx_hbm.at[2, pl.ds(cbase(g, b), K)], dv[b], sem[b]).wait()

        for b in range(G):
            issue_idx(0, b, srcA, dstA, isemA)
            issue_idx(1, b, srcB, dstB, isemB)
        for b in range(G):
            wait_idx(0, b, srcA, dstA, isemA)
            pltpu.async_copy(h_hbm.at[srcA[b]], rows_v[b], gsem[b])

        def subgroup(g, cur, nxt):
            (csrc, cdst, cisem), (nsrc, ndst, nisem) = cur, nxt
            scatters = []
            for b in range(G):
                pltpu.make_async_copy(
                    h_hbm.at[csrc[b]], rows_v[b], gsem[b]).wait()
                scatters.append(pltpu.async_copy(
                    rows_v[b], acc_sh.at[cdst[b]], ssem[b], add=True))
                i = g * G + b
                @pl.when(jnp.where(c == 0, i < half, i >= half))
                def _(b=b, i=i):
                    pltpu.async_copy(
                        ones_v, deg_sh.at[cdst[b]], dsem[b], add=True)

            for b in range(G):
                scatters[b].wait()
                i = g * G + b

                @pl.when(jnp.where(c == 0, i < half, i >= half))
                def _(b=b, i=i):
                    pltpu.make_async_copy(
                        ones_v, deg_sh.at[cdst[b]], dsem[b]).wait()

                @pl.when(g + 1 < ngroups)
                def _(b=b):
                    wait_idx(g + 1, b, nsrc, ndst, nisem)
                    pltpu.async_copy(h_hbm.at[nsrc[b]], rows_v[b], gsem[b])

                @pl.when(g + 2 < ngroups)
                def _(b=b):
                    issue_idx(g + 2, b, csrc, cdst, cisem)

        ringA = (srcA, dstA, isemA)
        ringB = (srcB, dstB, isemB)

        @pl.loop(0, ngroups // 2)
        def _(t):
            subgroup(2 * t, ringA, ringB)
            subgroup(2 * t + 1, ringB, ringA)

        plsc.subcore_barrier()

        pltpu.sync_copy(acc_sh.at[pl.ds(s * rpt, rpt)],
                        partial_hbm.at[pl.ds(s * rpt, rpt), pl.ds(c * dh, dh)])
        pltpu.sync_copy(deg_sh.at[pl.ds(s * rpt, rpt)],
                        deg_hbm.at[c, pl.ds(s * rpt, rpt)])

    return sc_kernel(hidden2, idx3)


def _mlp2(partial, degp, W_out, b_out, n, *, grid_n=2):
    d_hid = partial.shape[1]
    d_out = W_out.shape[1]
    blk = n // grid_n

    def body(p_ref, d_ref, w_ref, b_ref, o_ref):
        deg = d_ref[0][:, 0:1] + d_ref[1][:, 0:1]
        agg = p_ref[...] / jnp.maximum(deg, 1.0)
        o_ref[...] = (
            jnp.dot(agg, w_ref[...], preferred_element_type=jnp.float32)
            + b_ref[...]
        )

    return pl.pallas_call(
        body,
        grid=(grid_n,),
        in_specs=[
            pl.BlockSpec((blk, d_hid), lambda i: (i, 0)),
            pl.BlockSpec((NC, blk, LANES), lambda i: (0, i, 0)),
            pl.BlockSpec((d_hid, d_out), lambda i: (0, 0)),
            pl.BlockSpec((1, d_out), lambda i: (0, 0)),
        ],
        out_specs=pl.BlockSpec((blk, d_out), lambda i: (i, 0)),
        out_shape=jax.ShapeDtypeStruct((n, d_out), jnp.float32),
    )(partial, degp, W_out, b_out.reshape(1, d_out))


def kernel(node_features, edge_index, W1, b1, W_out, b_out):
    n, d_hid = node_features.shape[0], W1.shape[1]
    ei = edge_index.astype(jnp.int32)
    idx3 = jnp.stack([2 * ei[0], 2 * ei[0] + 1, ei[1]])
    hidden = _mlp1(node_features, W1, b1)
    hidden2 = hidden.reshape(NC * n, d_hid // NC)
    partial, degp = _sc_aggregate(hidden2, idx3)
    return _mlp2(partial, degp, W_out, b_out, n)
